# baseline (speedup 1.0000x reference)
import functools

import jax
import jax.numpy as jnp
from jax import lax
from jax.experimental import pallas as pl
from jax.experimental.pallas import tpu as pltpu
from jax.experimental.pallas import tpu_sc as plsc

N = 10000
E = 160000
G = 128
D = 256
H = 5000
NSC = 2
NT = 16
LN = 16
BLK = 1000
NBLK = N // BLK
EC = 128
RNG = N // NT
CAP = 8192
RCHUNK = 1600
ZROWS = 640

@functools.cache
def _mesh():
    return plsc.VectorSubcoreMesh(
        core_axis_name="c", subcore_axis_name="s",
        num_cores=NSC, num_subcores=NT)


def _route_body(src_hbm, dst_hbm, csrc_hbm, cdst_hbm, cnt_hbm, degp_hbm,
                csrc_v, cdst_v, ebuf_s, ebuf_d, ebuf_s2, ebuf_d2,
                deg_v, cnt_v, sem_a, sem_b):
    c = lax.axis_index("c")
    s = lax.axis_index("s")
    wid = c * NT + s
    lo = s * RNG
    ones = jnp.ones((LN,), jnp.float32)
    pad_dst = jnp.full((LN,), lo + RNG, jnp.int32)
    pad_src = jnp.zeros((LN,), jnp.int32)

    def zero_deg(i, _):
        deg_v[pl.ds(i * LN, LN)] = jnp.zeros((LN,), jnp.float32)
        return _
    lax.fori_loop(0, (N + LN) // LN, zero_deg, None)

    def prefill(i, _):
        csrc_v[pl.ds(i * LN, LN)] = pad_src
        cdst_v[pl.ds(i * LN, LN)] = pad_dst
        return _
    lax.fori_loop(0, CAP // LN, prefill, None)

    nchunks = (E // NSC) // RCHUNK

    def rstart(k, bs, bd, sem):
        @pl.when(k < nchunks)
        def _():
            base = c * (E // NSC) + k * RCHUNK
            pltpu.make_async_copy(src_hbm.at[pl.ds(base, RCHUNK)], bs, sem).start()
            pltpu.make_async_copy(dst_hbm.at[pl.ds(base, RCHUNK)], bd, sem).start()

    def rfin(k, bs, bd, sem, pos):
        base = c * (E // NSC) + k * RCHUNK
        pltpu.make_async_copy(src_hbm.at[pl.ds(base, RCHUNK)], bs, sem).wait()
        pltpu.make_async_copy(dst_hbm.at[pl.ds(base, RCHUNK)], bd, sem).wait()
        for g in range(RCHUNK // LN):
            sv = bs[pl.ds(g * LN, LN)]
            dv = bd[pl.ds(g * LN, LN)]
            m = (dv >= lo) & (dv < lo + RNG)
            plsc.addupdate_scatter(deg_v, [dv], ones, mask=m)
            p = jnp.minimum(pos, CAP - LN)
            plsc.store_compressed(csrc_v.at[pl.ds(p, LN)], sv, mask=m)
            plsc.store_compressed(cdst_v.at[pl.ds(p, LN)], dv, mask=m)
            pos = pos + jnp.sum(m.astype(jnp.int32), axis=0)
        return pos

    rstart(jnp.int32(0), ebuf_s, ebuf_d, sem_a)
    rstart(jnp.int32(1), ebuf_s2, ebuf_d2, sem_b)

    def pipe(i, pos):
        k0 = 2 * i
        pos = rfin(k0, ebuf_s, ebuf_d, sem_a, pos)
        rstart(k0 + 2, ebuf_s, ebuf_d, sem_a)
        pos = rfin(k0 + 1, ebuf_s2, ebuf_d2, sem_b, pos)
        rstart(k0 + 3, ebuf_s2, ebuf_d2, sem_b)
        return pos
    pos = lax.fori_loop(0, nchunks // 2, pipe, jnp.int32(0))

    cnt_v[pl.ds(0, LN)] = jnp.zeros((LN,), jnp.int32) + pos
    pltpu.sync_copy(csrc_v, csrc_hbm.at[wid])
    pltpu.sync_copy(cdst_v, cdst_hbm.at[wid])
    pltpu.sync_copy(cnt_v, cnt_hbm.at[wid])
    pltpu.sync_copy(deg_v, degp_hbm.at[wid])


@functools.cache
def _route_call():
    return pl.kernel(
        _route_body,
        out_type=[
            jax.ShapeDtypeStruct((NSC * NT, CAP), jnp.int32),
            jax.ShapeDtypeStruct((NSC * NT, CAP), jnp.int32),
            jax.ShapeDtypeStruct((NSC * NT, LN), jnp.int32),
            jax.ShapeDtypeStruct((NSC * NT, N + LN), jnp.float32),
        ],
        mesh=_mesh(),
        compiler_params=pltpu.CompilerParams(needs_layout_passes=False),
        scratch_types=[
            pltpu.VMEM((CAP,), jnp.int32),
            pltpu.VMEM((CAP,), jnp.int32),
            pltpu.VMEM((RCHUNK,), jnp.int32),
            pltpu.VMEM((RCHUNK,), jnp.int32),
            pltpu.VMEM((RCHUNK,), jnp.int32),
            pltpu.VMEM((RCHUNK,), jnp.int32),
            pltpu.VMEM((N + LN,), jnp.float32),
            pltpu.VMEM((LN,), jnp.int32),
            pltpu.SemaphoreType.DMA,
            pltpu.SemaphoreType.DMA,
        ],
    )


DH = D // 2
NGRP = EC // LN


def _conv_body(y128_hbm, csrc_hbm, cdst_hbm, cnt_hbm, z_hbm,
               zsl, gbuf0, gbuf1, csb0, csb1, cdb0, cdb1, cntbuf, sem0, sem1):
    c = lax.axis_index("c")
    s = lax.axis_index("s")
    lo = s * RNG
    iot = lax.broadcasted_iota(jnp.int32, (LN,), 0)
    zeros = jnp.zeros((LN,), jnp.float32)

    rots = [((iot + t) & (LN - 1)) for t in range(LN)]

    def zz(i, _):
        rows = jnp.full((LN,), i * LN, jnp.int32) + iot
        for r in range(DH // LN):
            for t in range(LN):
                plsc.store_scatter(zsl, [rows, rots[t] + r * LN], zeros)
        return _
    lax.fori_loop(0, ZROWS // LN, zz, None)

    pltpu.sync_copy(cnt_hbm.at[s], cntbuf)
    cnt0 = jnp.max(cntbuf[pl.ds(0, LN)], axis=0)
    pltpu.sync_copy(cnt_hbm.at[NT + s], cntbuf)
    cnt1 = jnp.max(cntbuf[pl.ds(0, LN)], axis=0)
    nch0 = (cnt0 + EC - 1) // EC
    ntot = nch0 + (cnt1 + EC - 1) // EC

    def stage_start(k, csb, cdb, gb, sem):
        @pl.when(k < ntot)
        def _():
            h = jnp.where(k >= nch0, 1, 0)
            row = h * NT + s
            eb = (k - h * nch0) * EC
            pltpu.sync_copy(csrc_hbm.at[row, pl.ds(eb, EC)], csb)
            pltpu.sync_copy(cdst_hbm.at[row, pl.ds(eb, EC)], cdb)
            for j in range(NGRP):
                v = csb[pl.ds(j * LN, LN)]
                csb[pl.ds(j * LN, LN)] = (v << 1) + c
                w = cdb[pl.ds(j * LN, LN)]
                cdb[pl.ds(j * LN, LN)] = w - lo
            pltpu.make_async_copy(y128_hbm.at[csb], gb, sem).start()

    def stage_fin(k, csb, cdb, gb, sem):
        @pl.when(k < ntot)
        def _():
            pltpu.make_async_copy(y128_hbm.at[csb], gb, sem).wait()

            def grp(g, _):
                erow = jnp.full((LN,), 0, jnp.int32) + g * LN + iot
                dvec = cdb[pl.ds(g * LN, LN)]
                for r in range(DH // LN):
                    for t in range(LN):
                        kcol = rots[t] + r * LN
                        vals = plsc.load_gather(gb, [erow, kcol])
                        plsc.addupdate_scatter(zsl, [dvec, kcol], vals)
                return _
            lax.fori_loop(0, NGRP, grp, None)

    stage_start(jnp.int32(0), csb0, cdb0, gbuf0, sem0)

    def pipe(i, _):
        k0 = 2 * i
        stage_start(k0 + 1, csb1, cdb1, gbuf1, sem1)
        stage_fin(k0, csb0, cdb0, gbuf0, sem0)
        stage_start(k0 + 2, csb0, cdb0, gbuf0, sem0)
        stage_fin(k0 + 1, csb1, cdb1, gbuf1, sem1)
        return _
    lax.fori_loop(0, (ntot + 1) // 2, pipe, None)

    pltpu.sync_copy(zsl.at[pl.ds(0, RNG)], z_hbm.at[pl.ds(lo, RNG), c])


@functools.cache
def _conv_call():
    return pl.kernel(
        _conv_body,
        out_type=jax.ShapeDtypeStruct((N, NSC, DH), jnp.float32),
        mesh=_mesh(),
        compiler_params=pltpu.CompilerParams(needs_layout_passes=False),
        scratch_types=[
            pltpu.VMEM((ZROWS, DH), jnp.float32),
            pltpu.VMEM((EC, DH), jnp.float32),
            pltpu.VMEM((EC, DH), jnp.float32),
            pltpu.VMEM((EC,), jnp.int32),
            pltpu.VMEM((EC,), jnp.int32),
            pltpu.VMEM((EC,), jnp.int32),
            pltpu.VMEM((EC,), jnp.int32),
            pltpu.VMEM((LN,), jnp.int32),
            pltpu.SemaphoreType.DMA,
            pltpu.SemaphoreType.DMA,
        ],
    )


def _tc1_body(x_ref, emb_ref, w1_ref, degp_ref, batch_ref,
              y_ref, dinv_ref, cnt_ref):
    i = pl.program_id(0)
    ew = jnp.dot(emb_ref[...], w1_ref[...],
                 preferred_element_type=jnp.float32)
    xb = x_ref[...][:, 0]
    oh = (xb[:, None] ==
          lax.broadcasted_iota(jnp.int32, (BLK, 28), 1)).astype(jnp.float32)
    deg = jnp.sum(degp_ref[0], axis=0) + 1.0
    dinv = lax.rsqrt(deg)
    y = jnp.dot(oh, ew, preferred_element_type=jnp.float32) * dinv[:, None]
    y_ref[...] = y
    dinv_ref[...] = dinv[:, None]

    bb = batch_ref[0, 0, :]
    g = (bb[:, None] ==
         lax.broadcasted_iota(jnp.int32, (BLK, G), 1)).astype(jnp.float32)
    cw = lax.dot_general(g, jnp.ones((BLK, 1), jnp.float32),
                         (((0,), (0,)), ((), ())))

    @pl.when(i == 0)
    def _():
        cnt_ref[...] = jnp.zeros((G, 1), jnp.float32)
    cnt_ref[...] += cw


def _tc1(x, emb_table, w1, degp, batch3):
    return pl.pallas_call(
        _tc1_body,
        grid=(NBLK,),
        in_specs=[
            pl.BlockSpec((BLK, 1), lambda i: (i, 0)),
            pl.BlockSpec((28, D), lambda i: (0, 0)),
            pl.BlockSpec((D, D), lambda i: (0, 0)),
            pl.BlockSpec((1, NSC * NT, BLK), lambda i: (i, 0, 0)),
            pl.BlockSpec((1, 1, BLK), lambda i: (i, 0, 0)),
        ],
        out_specs=[
            pl.BlockSpec((BLK, D), lambda i: (i, 0)),
            pl.BlockSpec((BLK, 1), lambda i: (i, 0)),
            pl.BlockSpec((G, 1), lambda i: (0, 0)),
        ],
        out_shape=[
            jax.ShapeDtypeStruct((N, D), jnp.float32),
            jax.ShapeDtypeStruct((N, 1), jnp.float32),
            jax.ShapeDtypeStruct((G, 1), jnp.float32),
        ],
    )(x, emb_table, w1, degp, batch3)


def _tc2_body(z_ref, y_ref, dinv_ref, b_ref, w_ref, o_ref):
    h = jnp.maximum(
        (z_ref[...] + y_ref[...]) * dinv_ref[...] + b_ref[...], 0.0)
    o_ref[...] = jnp.dot(h, w_ref[...],
                         preferred_element_type=jnp.float32) * dinv_ref[...]


def _tc2(z1, y1, dinv, b1, w2):
    return pl.pallas_call(
        _tc2_body,
        grid=(NBLK,),
        in_specs=[
            pl.BlockSpec((BLK, D), lambda i: (i, 0)),
            pl.BlockSpec((BLK, D), lambda i: (i, 0)),
            pl.BlockSpec((BLK, 1), lambda i: (i, 0)),
            pl.BlockSpec((1, D), lambda i: (0, 0)),
            pl.BlockSpec((D, D), lambda i: (0, 0)),
        ],
        out_specs=pl.BlockSpec((BLK, D), lambda i: (i, 0)),
        out_shape=jax.ShapeDtypeStruct((N, D), jnp.float32),
    )(z1, y1, dinv, b1, w2)


def _tc3_body(z_ref, y_ref, dinv_ref, b_ref, batch_ref, cnt_ref,
              wl1_ref, bl1_ref, wl2_ref, bl2_ref, o_ref, acc_ref):
    i = pl.program_id(0)
    emb = jnp.maximum(
        (z_ref[...] + y_ref[...]) * dinv_ref[...] + b_ref[...], 0.0)
    bb = batch_ref[0, 0, :]
    g = (bb[:, None] ==
         lax.broadcasted_iota(jnp.int32, (BLK, G), 1)).astype(jnp.float32)
    part = lax.dot_general(g, emb, (((0,), (0,)), ((), ())))

    @pl.when(i == 0)
    def _():
        acc_ref[...] = jnp.zeros((G, D), jnp.float32)
    acc_ref[...] += part

    @pl.when(i == NBLK - 1)
    def _():
        cnt = jnp.maximum(cnt_ref[...], 1.0)
        pooled = jnp.maximum(acc_ref[...] / cnt, 0.0)
        hh = jnp.maximum(
            jnp.dot(pooled, wl1_ref[...],
                    preferred_element_type=jnp.float32) + bl1_ref[...], 0.0)
        o_ref[...] = jnp.dot(hh, wl2_ref[...],
                             preferred_element_type=jnp.float32) + bl2_ref[...]


def _tc3(z2, y2, dinv, b2, batch3, cnt, wl1, bl1, wl2, bl2):
    return pl.pallas_call(
        _tc3_body,
        grid=(NBLK,),
        in_specs=[
            pl.BlockSpec((BLK, D), lambda i: (i, 0)),
            pl.BlockSpec((BLK, D), lambda i: (i, 0)),
            pl.BlockSpec((BLK, 1), lambda i: (i, 0)),
            pl.BlockSpec((1, D), lambda i: (0, 0)),
            pl.BlockSpec((1, 1, BLK), lambda i: (i, 0, 0)),
            pl.BlockSpec((G, 1), lambda i: (0, 0)),
            pl.BlockSpec((D, 64), lambda i: (0, 0)),
            pl.BlockSpec((1, 64), lambda i: (0, 0)),
            pl.BlockSpec((64, 1), lambda i: (0, 0)),
            pl.BlockSpec((1, 1), lambda i: (0, 0)),
        ],
        out_specs=pl.BlockSpec((G, 1), lambda i: (0, 0)),
        out_shape=jax.ShapeDtypeStruct((G, 1), jnp.float32),
        scratch_shapes=[pltpu.VMEM((G, D), jnp.float32)],
    )(z2, y2, dinv, b2, batch3, cnt, wl1, bl1, wl2, bl2)


@jax.jit
def kernel(x, edge_index, edge_attr, batch, emb_table,
           W1, b1, W2, b2, Wl1, bl1, Wl2, bl2):
    del edge_attr
    src = edge_index[0]
    dst = edge_index[1]
    batch3 = batch.reshape(NBLK, 1, BLK)

    csrc, cdst_r, cnts, degp_raw = _route_call()(src, dst)
    degp = degp_raw[:, :N].reshape(NSC * NT, NBLK, BLK).transpose(1, 0, 2)

    y1, dinv, cnt = _tc1(x, emb_table, W1, degp, batch3)
    z1 = _conv_call()(y1.reshape(N * NSC, DH), csrc, cdst_r, cnts).reshape(N, D)
    y2 = _tc2(z1, y1, dinv, b1.reshape(1, D), W2)
    z2 = _conv_call()(y2.reshape(N * NSC, DH), csrc, cdst_r, cnts).reshape(N, D)
    out = _tc3(z2, y2, dinv, b2.reshape(1, D), batch3, cnt,
               Wl1, bl1.reshape(1, 64), Wl2, bl2.reshape(1, 1))
    return out[:, 0]


# speedup vs baseline: 1.2850x; 1.2850x over previous
import functools

import jax
import jax.numpy as jnp
from jax import lax
from jax.experimental import pallas as pl
from jax.experimental.pallas import tpu as pltpu
from jax.experimental.pallas import tpu_sc as plsc

N = 10000
E = 160000
G = 128
D = 256
H = 5000
NSC = 2
NT = 16
LN = 16
BLK = 1000
NBLK = N // BLK
EC = 128
RNG = N // NT
CAP = 8192
RCHUNK = 1600
ZROWS = 640

@functools.cache
def _mesh():
    return plsc.VectorSubcoreMesh(
        core_axis_name="c", subcore_axis_name="s",
        num_cores=NSC, num_subcores=NT)


def _route_body(src_hbm, dst_hbm, csrc_hbm, cdst_hbm, cnt_hbm, degp_hbm,
                csrc_v, cdst_v, ebuf_s, ebuf_d, ebuf_s2, ebuf_d2,
                deg_v, cnt_v, sem_a, sem_b):
    c = lax.axis_index("c")
    s = lax.axis_index("s")
    wid = c * NT + s
    lo = s * RNG
    ones = jnp.ones((LN,), jnp.float32)
    pad_dst = jnp.full((LN,), lo + RNG, jnp.int32)
    pad_src = jnp.zeros((LN,), jnp.int32)

    def zero_deg(i, _):
        deg_v[pl.ds(i * LN, LN)] = jnp.zeros((LN,), jnp.float32)
        return _
    lax.fori_loop(0, (N + LN) // LN, zero_deg, None)

    def prefill(i, _):
        csrc_v[pl.ds(i * LN, LN)] = pad_src
        cdst_v[pl.ds(i * LN, LN)] = pad_dst
        return _
    lax.fori_loop(0, CAP // LN, prefill, None)

    nchunks = (E // NSC) // RCHUNK

    def rstart(k, bs, bd, sem):
        @pl.when(k < nchunks)
        def _():
            base = c * (E // NSC) + k * RCHUNK
            pltpu.make_async_copy(src_hbm.at[pl.ds(base, RCHUNK)], bs, sem).start()
            pltpu.make_async_copy(dst_hbm.at[pl.ds(base, RCHUNK)], bd, sem).start()

    def rfin(k, bs, bd, sem, pos):
        base = c * (E // NSC) + k * RCHUNK
        pltpu.make_async_copy(src_hbm.at[pl.ds(base, RCHUNK)], bs, sem).wait()
        pltpu.make_async_copy(dst_hbm.at[pl.ds(base, RCHUNK)], bd, sem).wait()
        for g in range(RCHUNK // LN):
            sv = bs[pl.ds(g * LN, LN)]
            dv = bd[pl.ds(g * LN, LN)]
            m = (dv >= lo) & (dv < lo + RNG)
            plsc.addupdate_scatter(deg_v, [dv], ones, mask=m)
            p = jnp.minimum(pos, CAP - LN)
            plsc.store_compressed(csrc_v.at[pl.ds(p, LN)], sv, mask=m)
            plsc.store_compressed(cdst_v.at[pl.ds(p, LN)], dv, mask=m)
            pos = pos + jnp.sum(m.astype(jnp.int32), axis=0)
        return pos

    rstart(jnp.int32(0), ebuf_s, ebuf_d, sem_a)
    rstart(jnp.int32(1), ebuf_s2, ebuf_d2, sem_b)

    def pipe(i, pos):
        k0 = 2 * i
        pos = rfin(k0, ebuf_s, ebuf_d, sem_a, pos)
        rstart(k0 + 2, ebuf_s, ebuf_d, sem_a)
        pos = rfin(k0 + 1, ebuf_s2, ebuf_d2, sem_b, pos)
        rstart(k0 + 3, ebuf_s2, ebuf_d2, sem_b)
        return pos
    pos = lax.fori_loop(0, nchunks // 2, pipe, jnp.int32(0))

    cnt_v[pl.ds(0, LN)] = jnp.zeros((LN,), jnp.int32) + pos
    pltpu.sync_copy(csrc_v, csrc_hbm.at[wid])
    pltpu.sync_copy(cdst_v, cdst_hbm.at[wid])
    pltpu.sync_copy(cnt_v, cnt_hbm.at[wid])
    pltpu.sync_copy(deg_v, degp_hbm.at[wid])


@functools.cache
def _route_call():
    return pl.kernel(
        _route_body,
        out_type=[
            jax.ShapeDtypeStruct((NSC * NT, CAP), jnp.int32),
            jax.ShapeDtypeStruct((NSC * NT, CAP), jnp.int32),
            jax.ShapeDtypeStruct((NSC * NT, LN), jnp.int32),
            jax.ShapeDtypeStruct((NSC * NT, N + LN), jnp.float32),
        ],
        mesh=_mesh(),
        compiler_params=pltpu.CompilerParams(needs_layout_passes=False),
        scratch_types=[
            pltpu.VMEM((CAP,), jnp.int32),
            pltpu.VMEM((CAP,), jnp.int32),
            pltpu.VMEM((RCHUNK,), jnp.int32),
            pltpu.VMEM((RCHUNK,), jnp.int32),
            pltpu.VMEM((RCHUNK,), jnp.int32),
            pltpu.VMEM((RCHUNK,), jnp.int32),
            pltpu.VMEM((N + LN,), jnp.float32),
            pltpu.VMEM((LN,), jnp.int32),
            pltpu.SemaphoreType.DMA,
            pltpu.SemaphoreType.DMA,
        ],
    )


DH = D // 2
NGRP = EC // LN


def _conv_body(y128_hbm, csrc_hbm, cdst_hbm, cnt_hbm, z_hbm,
               zsl, gbuf0, gbuf1, csb0, csb1, cdb0, cdb1, cntbuf, sem0, sem1):
    c = lax.axis_index("c")
    s = lax.axis_index("s")
    lo = s * RNG
    iot = lax.broadcasted_iota(jnp.int32, (LN,), 0)
    zeros = jnp.zeros((LN,), jnp.float32)

    rots = [((iot + t) & (LN - 1)) for t in range(LN)]

    def zz(i, _):
        rows = jnp.full((LN,), i * LN, jnp.int32) + iot
        for r in range(DH // LN):
            for t in range(LN):
                plsc.store_scatter(zsl, [rows, rots[t] + r * LN], zeros)
        return _
    lax.fori_loop(0, ZROWS // LN, zz, None)

    pltpu.sync_copy(cnt_hbm.at[s], cntbuf)
    cnt0 = jnp.max(cntbuf[pl.ds(0, LN)], axis=0)
    pltpu.sync_copy(cnt_hbm.at[NT + s], cntbuf)
    cnt1 = jnp.max(cntbuf[pl.ds(0, LN)], axis=0)
    nch0 = (cnt0 + EC - 1) // EC
    ntot = nch0 + (cnt1 + EC - 1) // EC

    def stage_start(k, csb, cdb, gb, sem):
        @pl.when(k < ntot)
        def _():
            h = jnp.where(k >= nch0, 1, 0)
            row = h * NT + s
            eb = (k - h * nch0) * EC
            pltpu.sync_copy(csrc_hbm.at[row, pl.ds(eb, EC)], csb)
            pltpu.sync_copy(cdst_hbm.at[row, pl.ds(eb, EC)], cdb)
            for j in range(NGRP):
                v = csb[pl.ds(j * LN, LN)]
                csb[pl.ds(j * LN, LN)] = (v << 1) + c
                w = cdb[pl.ds(j * LN, LN)]
                cdb[pl.ds(j * LN, LN)] = w - lo
            pltpu.make_async_copy(y128_hbm.at[csb], gb, sem).start()

    def stage_fin(k, csb, cdb, gb, sem):
        @pl.when(k < ntot)
        def _():
            pltpu.make_async_copy(y128_hbm.at[csb], gb, sem).wait()

            @plsc.parallel_loop(0, NGRP, unroll=2)
            def grp(g):
                erow = jnp.full((LN,), 0, jnp.int32) + g * LN + iot
                dvec = cdb[pl.ds(g * LN, LN)]
                for r in range(DH // LN):
                    for t in range(LN):
                        kcol = rots[t] + r * LN
                        vals = plsc.load_gather(gb, [erow, kcol])
                        plsc.addupdate_scatter(zsl, [dvec, kcol], vals)

    stage_start(jnp.int32(0), csb0, cdb0, gbuf0, sem0)

    def pipe(i, _):
        k0 = 2 * i
        stage_start(k0 + 1, csb1, cdb1, gbuf1, sem1)
        stage_fin(k0, csb0, cdb0, gbuf0, sem0)
        stage_start(k0 + 2, csb0, cdb0, gbuf0, sem0)
        stage_fin(k0 + 1, csb1, cdb1, gbuf1, sem1)
        return _
    lax.fori_loop(0, (ntot + 1) // 2, pipe, None)

    pltpu.sync_copy(zsl.at[pl.ds(0, RNG)], z_hbm.at[pl.ds(lo, RNG), c])


@functools.cache
def _conv_call():
    return pl.kernel(
        _conv_body,
        out_type=jax.ShapeDtypeStruct((N, NSC, DH), jnp.float32),
        mesh=_mesh(),
        compiler_params=pltpu.CompilerParams(needs_layout_passes=False),
        scratch_types=[
            pltpu.VMEM((ZROWS, DH), jnp.float32),
            pltpu.VMEM((EC, DH), jnp.float32),
            pltpu.VMEM((EC, DH), jnp.float32),
            pltpu.VMEM((EC,), jnp.int32),
            pltpu.VMEM((EC,), jnp.int32),
            pltpu.VMEM((EC,), jnp.int32),
            pltpu.VMEM((EC,), jnp.int32),
            pltpu.VMEM((LN,), jnp.int32),
            pltpu.SemaphoreType.DMA,
            pltpu.SemaphoreType.DMA,
        ],
    )


def _tc1_body(x_ref, emb_ref, w1_ref, degp_ref, batch_ref,
              y_ref, dinv_ref, cnt_ref):
    i = pl.program_id(0)
    ew = jnp.dot(emb_ref[...], w1_ref[...],
                 preferred_element_type=jnp.float32)
    xb = x_ref[...][:, 0]
    oh = (xb[:, None] ==
          lax.broadcasted_iota(jnp.int32, (BLK, 28), 1)).astype(jnp.float32)
    deg = jnp.sum(degp_ref[0], axis=0) + 1.0
    dinv = lax.rsqrt(deg)
    y = jnp.dot(oh, ew, preferred_element_type=jnp.float32) * dinv[:, None]
    y_ref[...] = y
    dinv_ref[...] = dinv[:, None]

    bb = batch_ref[0, 0, :]
    g = (bb[:, None] ==
         lax.broadcasted_iota(jnp.int32, (BLK, G), 1)).astype(jnp.float32)
    cw = lax.dot_general(g, jnp.ones((BLK, 1), jnp.float32),
                         (((0,), (0,)), ((), ())))

    @pl.when(i == 0)
    def _():
        cnt_ref[...] = jnp.zeros((G, 1), jnp.float32)
    cnt_ref[...] += cw


def _tc1(x, emb_table, w1, degp, batch3):
    return pl.pallas_call(
        _tc1_body,
        grid=(NBLK,),
        in_specs=[
            pl.BlockSpec((BLK, 1), lambda i: (i, 0)),
            pl.BlockSpec((28, D), lambda i: (0, 0)),
            pl.BlockSpec((D, D), lambda i: (0, 0)),
            pl.BlockSpec((1, NSC * NT, BLK), lambda i: (i, 0, 0)),
            pl.BlockSpec((1, 1, BLK), lambda i: (i, 0, 0)),
        ],
        out_specs=[
            pl.BlockSpec((BLK, D), lambda i: (i, 0)),
            pl.BlockSpec((BLK, 1), lambda i: (i, 0)),
            pl.BlockSpec((G, 1), lambda i: (0, 0)),
        ],
        out_shape=[
            jax.ShapeDtypeStruct((N, D), jnp.float32),
            jax.ShapeDtypeStruct((N, 1), jnp.float32),
            jax.ShapeDtypeStruct((G, 1), jnp.float32),
        ],
    )(x, emb_table, w1, degp, batch3)


def _tc2_body(z_ref, y_ref, dinv_ref, b_ref, w_ref, o_ref):
    h = jnp.maximum(
        (z_ref[...] + y_ref[...]) * dinv_ref[...] + b_ref[...], 0.0)
    o_ref[...] = jnp.dot(h, w_ref[...],
                         preferred_element_type=jnp.float32) * dinv_ref[...]


def _tc2(z1, y1, dinv, b1, w2):
    return pl.pallas_call(
        _tc2_body,
        grid=(NBLK,),
        in_specs=[
            pl.BlockSpec((BLK, D), lambda i: (i, 0)),
            pl.BlockSpec((BLK, D), lambda i: (i, 0)),
            pl.BlockSpec((BLK, 1), lambda i: (i, 0)),
            pl.BlockSpec((1, D), lambda i: (0, 0)),
            pl.BlockSpec((D, D), lambda i: (0, 0)),
        ],
        out_specs=pl.BlockSpec((BLK, D), lambda i: (i, 0)),
        out_shape=jax.ShapeDtypeStruct((N, D), jnp.float32),
    )(z1, y1, dinv, b1, w2)


def _tc3_body(z_ref, y_ref, dinv_ref, b_ref, batch_ref, cnt_ref,
              wl1_ref, bl1_ref, wl2_ref, bl2_ref, o_ref, acc_ref):
    i = pl.program_id(0)
    emb = jnp.maximum(
        (z_ref[...] + y_ref[...]) * dinv_ref[...] + b_ref[...], 0.0)
    bb = batch_ref[0, 0, :]
    g = (bb[:, None] ==
         lax.broadcasted_iota(jnp.int32, (BLK, G), 1)).astype(jnp.float32)
    part = lax.dot_general(g, emb, (((0,), (0,)), ((), ())))

    @pl.when(i == 0)
    def _():
        acc_ref[...] = jnp.zeros((G, D), jnp.float32)
    acc_ref[...] += part

    @pl.when(i == NBLK - 1)
    def _():
        cnt = jnp.maximum(cnt_ref[...], 1.0)
        pooled = jnp.maximum(acc_ref[...] / cnt, 0.0)
        hh = jnp.maximum(
            jnp.dot(pooled, wl1_ref[...],
                    preferred_element_type=jnp.float32) + bl1_ref[...], 0.0)
        o_ref[...] = jnp.dot(hh, wl2_ref[...],
                             preferred_element_type=jnp.float32) + bl2_ref[...]


def _tc3(z2, y2, dinv, b2, batch3, cnt, wl1, bl1, wl2, bl2):
    return pl.pallas_call(
        _tc3_body,
        grid=(NBLK,),
        in_specs=[
            pl.BlockSpec((BLK, D), lambda i: (i, 0)),
            pl.BlockSpec((BLK, D), lambda i: (i, 0)),
            pl.BlockSpec((BLK, 1), lambda i: (i, 0)),
            pl.BlockSpec((1, D), lambda i: (0, 0)),
            pl.BlockSpec((1, 1, BLK), lambda i: (i, 0, 0)),
            pl.BlockSpec((G, 1), lambda i: (0, 0)),
            pl.BlockSpec((D, 64), lambda i: (0, 0)),
            pl.BlockSpec((1, 64), lambda i: (0, 0)),
            pl.BlockSpec((64, 1), lambda i: (0, 0)),
            pl.BlockSpec((1, 1), lambda i: (0, 0)),
        ],
        out_specs=pl.BlockSpec((G, 1), lambda i: (0, 0)),
        out_shape=jax.ShapeDtypeStruct((G, 1), jnp.float32),
        scratch_shapes=[pltpu.VMEM((G, D), jnp.float32)],
    )(z2, y2, dinv, b2, batch3, cnt, wl1, bl1, wl2, bl2)


@jax.jit
def kernel(x, edge_index, edge_attr, batch, emb_table,
           W1, b1, W2, b2, Wl1, bl1, Wl2, bl2):
    del edge_attr
    src = edge_index[0]
    dst = edge_index[1]
    batch3 = batch.reshape(NBLK, 1, BLK)

    csrc, cdst_r, cnts, degp_raw = _route_call()(src, dst)
    degp = degp_raw[:, :N].reshape(NSC * NT, NBLK, BLK).transpose(1, 0, 2)

    y1, dinv, cnt = _tc1(x, emb_table, W1, degp, batch3)
    z1 = _conv_call()(y1.reshape(N * NSC, DH), csrc, cdst_r, cnts).reshape(N, D)
    y2 = _tc2(z1, y1, dinv, b1.reshape(1, D), W2)
    z2 = _conv_call()(y2.reshape(N * NSC, DH), csrc, cdst_r, cnts).reshape(N, D)
    out = _tc3(z2, y2, dinv, b2.reshape(1, D), batch3, cnt,
               Wl1, bl1.reshape(1, 64), Wl2, bl2.reshape(1, 1))
    return out[:, 0]


# speedup vs baseline: 1.3808x; 1.0745x over previous
import functools

import jax
import jax.numpy as jnp
from jax import lax
from jax.experimental import pallas as pl
from jax.experimental.pallas import tpu as pltpu
from jax.experimental.pallas import tpu_sc as plsc

N = 10000
E = 160000
G = 128
D = 256
H = 5000
NSC = 2
NT = 16
LN = 16
BLK = 1000
NBLK = N // BLK
EC = 128
RNG = N // NT
CAP = 8192
RCHUNK = 1600
ZROWS = 640

@functools.cache
def _mesh():
    return plsc.VectorSubcoreMesh(
        core_axis_name="c", subcore_axis_name="s",
        num_cores=NSC, num_subcores=NT)


def _route_body(src_hbm, dst_hbm, csrc_hbm, cdst_hbm, cnt_hbm, degp_hbm,
                csrc_v, cdst_v, ebuf_s, ebuf_d, ebuf_s2, ebuf_d2,
                deg_v, cnt_v, sem_a, sem_b):
    c = lax.axis_index("c")
    s = lax.axis_index("s")
    wid = c * NT + s
    lo = s * RNG
    ones = jnp.ones((LN,), jnp.float32)
    pad_dst = jnp.full((LN,), lo + RNG, jnp.int32)
    pad_src = jnp.zeros((LN,), jnp.int32)

    def zero_deg(i, _):
        deg_v[pl.ds(i * LN, LN)] = jnp.zeros((LN,), jnp.float32)
        return _
    lax.fori_loop(0, (N + LN) // LN, zero_deg, None)

    def prefill(i, _):
        csrc_v[pl.ds(i * LN, LN)] = pad_src
        cdst_v[pl.ds(i * LN, LN)] = pad_dst
        return _
    lax.fori_loop(0, CAP // LN, prefill, None)

    nchunks = (E // NSC) // RCHUNK

    def rstart(k, bs, bd, sem):
        @pl.when(k < nchunks)
        def _():
            base = c * (E // NSC) + k * RCHUNK
            pltpu.make_async_copy(src_hbm.at[pl.ds(base, RCHUNK)], bs, sem).start()
            pltpu.make_async_copy(dst_hbm.at[pl.ds(base, RCHUNK)], bd, sem).start()

    def rfin(k, bs, bd, sem, pos):
        base = c * (E // NSC) + k * RCHUNK
        pltpu.make_async_copy(src_hbm.at[pl.ds(base, RCHUNK)], bs, sem).wait()
        pltpu.make_async_copy(dst_hbm.at[pl.ds(base, RCHUNK)], bd, sem).wait()
        @plsc.parallel_loop(0, RCHUNK // LN, unroll=2, carry=pos)
        def scan(g, pos):
            sv = bs[pl.ds(g * LN, LN)]
            dv = bd[pl.ds(g * LN, LN)]
            m = (dv >= lo) & (dv < lo + RNG)
            plsc.addupdate_scatter(deg_v, [dv], ones, mask=m)
            p = jnp.minimum(pos, CAP - LN)
            plsc.store_compressed(csrc_v.at[pl.ds(p, LN)], sv, mask=m)
            plsc.store_compressed(cdst_v.at[pl.ds(p, LN)], dv, mask=m)
            return pos + jnp.sum(m.astype(jnp.int32), axis=0)
        return scan

    rstart(jnp.int32(0), ebuf_s, ebuf_d, sem_a)
    rstart(jnp.int32(1), ebuf_s2, ebuf_d2, sem_b)

    def pipe(i, pos):
        k0 = 2 * i
        pos = rfin(k0, ebuf_s, ebuf_d, sem_a, pos)
        rstart(k0 + 2, ebuf_s, ebuf_d, sem_a)
        pos = rfin(k0 + 1, ebuf_s2, ebuf_d2, sem_b, pos)
        rstart(k0 + 3, ebuf_s2, ebuf_d2, sem_b)
        return pos
    pos = lax.fori_loop(0, nchunks // 2, pipe, jnp.int32(0))

    cnt_v[pl.ds(0, LN)] = jnp.zeros((LN,), jnp.int32) + pos
    pltpu.sync_copy(csrc_v, csrc_hbm.at[wid])
    pltpu.sync_copy(cdst_v, cdst_hbm.at[wid])
    pltpu.sync_copy(cnt_v, cnt_hbm.at[wid])
    pltpu.sync_copy(deg_v, degp_hbm.at[wid])


@functools.cache
def _route_call():
    return pl.kernel(
        _route_body,
        out_type=[
            jax.ShapeDtypeStruct((NSC * NT, CAP), jnp.int32),
            jax.ShapeDtypeStruct((NSC * NT, CAP), jnp.int32),
            jax.ShapeDtypeStruct((NSC * NT, LN), jnp.int32),
            jax.ShapeDtypeStruct((NSC * NT, N + LN), jnp.float32),
        ],
        mesh=_mesh(),
        compiler_params=pltpu.CompilerParams(needs_layout_passes=False),
        scratch_types=[
            pltpu.VMEM((CAP,), jnp.int32),
            pltpu.VMEM((CAP,), jnp.int32),
            pltpu.VMEM((RCHUNK,), jnp.int32),
            pltpu.VMEM((RCHUNK,), jnp.int32),
            pltpu.VMEM((RCHUNK,), jnp.int32),
            pltpu.VMEM((RCHUNK,), jnp.int32),
            pltpu.VMEM((N + LN,), jnp.float32),
            pltpu.VMEM((LN,), jnp.int32),
            pltpu.SemaphoreType.DMA,
            pltpu.SemaphoreType.DMA,
        ],
    )


DH = D // 2
NGRP = EC // LN


def _conv_body(y128_hbm, csrc_hbm, cdst_hbm, cnt_hbm, z_hbm,
               zsl, gbuf0, gbuf1, csb0, csb1, cdb0, cdb1, cntbuf, sem0, sem1):
    c = lax.axis_index("c")
    s = lax.axis_index("s")
    lo = s * RNG
    iot = lax.broadcasted_iota(jnp.int32, (LN,), 0)
    zeros = jnp.zeros((LN,), jnp.float32)

    rots = [((iot + t) & (LN - 1)) for t in range(LN)]

    @plsc.parallel_loop(0, ZROWS // LN, unroll=2)
    def zz(i):
        rows = jnp.full((LN,), i * LN, jnp.int32) + iot
        for r in range(DH // LN):
            for t in range(LN):
                plsc.store_scatter(zsl, [rows, rots[t] + r * LN], zeros)

    pltpu.sync_copy(cnt_hbm.at[s], cntbuf)
    cnt0 = jnp.max(cntbuf[pl.ds(0, LN)], axis=0)
    pltpu.sync_copy(cnt_hbm.at[NT + s], cntbuf)
    cnt1 = jnp.max(cntbuf[pl.ds(0, LN)], axis=0)
    nch0 = (cnt0 + EC - 1) // EC
    ntot = nch0 + (cnt1 + EC - 1) // EC

    def stage_start(k, csb, cdb, gb, sem):
        @pl.when(k < ntot)
        def _():
            h = jnp.where(k >= nch0, 1, 0)
            row = h * NT + s
            eb = (k - h * nch0) * EC
            pltpu.sync_copy(csrc_hbm.at[row, pl.ds(eb, EC)], csb)
            pltpu.sync_copy(cdst_hbm.at[row, pl.ds(eb, EC)], cdb)
            for j in range(NGRP):
                v = csb[pl.ds(j * LN, LN)]
                csb[pl.ds(j * LN, LN)] = (v << 1) + c
                w = cdb[pl.ds(j * LN, LN)]
                cdb[pl.ds(j * LN, LN)] = w - lo
            pltpu.make_async_copy(y128_hbm.at[csb], gb, sem).start()

    def stage_fin(k, csb, cdb, gb, sem):
        @pl.when(k < ntot)
        def _():
            pltpu.make_async_copy(y128_hbm.at[csb], gb, sem).wait()

            @plsc.parallel_loop(0, NGRP, unroll=2)
            def grp(g):
                erow = jnp.full((LN,), 0, jnp.int32) + g * LN + iot
                dvec = cdb[pl.ds(g * LN, LN)]
                for r in range(DH // LN):
                    for t in range(LN):
                        kcol = rots[t] + r * LN
                        vals = plsc.load_gather(gb, [erow, kcol])
                        plsc.addupdate_scatter(zsl, [dvec, kcol], vals)

    stage_start(jnp.int32(0), csb0, cdb0, gbuf0, sem0)

    def pipe(i, _):
        k0 = 2 * i
        stage_start(k0 + 1, csb1, cdb1, gbuf1, sem1)
        stage_fin(k0, csb0, cdb0, gbuf0, sem0)
        stage_start(k0 + 2, csb0, cdb0, gbuf0, sem0)
        stage_fin(k0 + 1, csb1, cdb1, gbuf1, sem1)
        return _
    lax.fori_loop(0, (ntot + 1) // 2, pipe, None)

    pltpu.sync_copy(zsl.at[pl.ds(0, RNG)], z_hbm.at[pl.ds(lo, RNG), c])


@functools.cache
def _conv_call():
    return pl.kernel(
        _conv_body,
        out_type=jax.ShapeDtypeStruct((N, NSC, DH), jnp.float32),
        mesh=_mesh(),
        compiler_params=pltpu.CompilerParams(needs_layout_passes=False),
        scratch_types=[
            pltpu.VMEM((ZROWS, DH), jnp.float32),
            pltpu.VMEM((EC, DH), jnp.float32),
            pltpu.VMEM((EC, DH), jnp.float32),
            pltpu.VMEM((EC,), jnp.int32),
            pltpu.VMEM((EC,), jnp.int32),
            pltpu.VMEM((EC,), jnp.int32),
            pltpu.VMEM((EC,), jnp.int32),
            pltpu.VMEM((LN,), jnp.int32),
            pltpu.SemaphoreType.DMA,
            pltpu.SemaphoreType.DMA,
        ],
    )


def _tc1_body(x_ref, emb_ref, w1_ref, degp_ref, batch_ref,
              y_ref, dinv_ref, cnt_ref):
    i = pl.program_id(0)
    ew = jnp.dot(emb_ref[...], w1_ref[...],
                 preferred_element_type=jnp.float32)
    xb = x_ref[...][:, 0]
    oh = (xb[:, None] ==
          lax.broadcasted_iota(jnp.int32, (BLK, 28), 1)).astype(jnp.float32)
    deg = jnp.sum(degp_ref[0], axis=0) + 1.0
    dinv = lax.rsqrt(deg)
    y = jnp.dot(oh, ew, preferred_element_type=jnp.float32) * dinv[:, None]
    y_ref[...] = y
    dinv_ref[...] = dinv[:, None]

    bb = batch_ref[0, 0, :]
    g = (bb[:, None] ==
         lax.broadcasted_iota(jnp.int32, (BLK, G), 1)).astype(jnp.float32)
    cw = lax.dot_general(g, jnp.ones((BLK, 1), jnp.float32),
                         (((0,), (0,)), ((), ())))

    @pl.when(i == 0)
    def _():
        cnt_ref[...] = jnp.zeros((G, 1), jnp.float32)
    cnt_ref[...] += cw


def _tc1(x, emb_table, w1, degp, batch3):
    return pl.pallas_call(
        _tc1_body,
        grid=(NBLK,),
        in_specs=[
            pl.BlockSpec((BLK, 1), lambda i: (i, 0)),
            pl.BlockSpec((28, D), lambda i: (0, 0)),
            pl.BlockSpec((D, D), lambda i: (0, 0)),
            pl.BlockSpec((1, NSC * NT, BLK), lambda i: (i, 0, 0)),
            pl.BlockSpec((1, 1, BLK), lambda i: (i, 0, 0)),
        ],
        out_specs=[
            pl.BlockSpec((BLK, D), lambda i: (i, 0)),
            pl.BlockSpec((BLK, 1), lambda i: (i, 0)),
            pl.BlockSpec((G, 1), lambda i: (0, 0)),
        ],
        out_shape=[
            jax.ShapeDtypeStruct((N, D), jnp.float32),
            jax.ShapeDtypeStruct((N, 1), jnp.float32),
            jax.ShapeDtypeStruct((G, 1), jnp.float32),
        ],
    )(x, emb_table, w1, degp, batch3)


def _tc2_body(z_ref, y_ref, dinv_ref, b_ref, w_ref, o_ref):
    h = jnp.maximum(
        (z_ref[...] + y_ref[...]) * dinv_ref[...] + b_ref[...], 0.0)
    o_ref[...] = jnp.dot(h, w_ref[...],
                         preferred_element_type=jnp.float32) * dinv_ref[...]


def _tc2(z1, y1, dinv, b1, w2):
    return pl.pallas_call(
        _tc2_body,
        grid=(NBLK,),
        in_specs=[
            pl.BlockSpec((BLK, D), lambda i: (i, 0)),
            pl.BlockSpec((BLK, D), lambda i: (i, 0)),
            pl.BlockSpec((BLK, 1), lambda i: (i, 0)),
            pl.BlockSpec((1, D), lambda i: (0, 0)),
            pl.BlockSpec((D, D), lambda i: (0, 0)),
        ],
        out_specs=pl.BlockSpec((BLK, D), lambda i: (i, 0)),
        out_shape=jax.ShapeDtypeStruct((N, D), jnp.float32),
    )(z1, y1, dinv, b1, w2)


def _tc3_body(z_ref, y_ref, dinv_ref, b_ref, batch_ref, cnt_ref,
              wl1_ref, bl1_ref, wl2_ref, bl2_ref, o_ref, acc_ref):
    i = pl.program_id(0)
    emb = jnp.maximum(
        (z_ref[...] + y_ref[...]) * dinv_ref[...] + b_ref[...], 0.0)
    bb = batch_ref[0, 0, :]
    g = (bb[:, None] ==
         lax.broadcasted_iota(jnp.int32, (BLK, G), 1)).astype(jnp.float32)
    part = lax.dot_general(g, emb, (((0,), (0,)), ((), ())))

    @pl.when(i == 0)
    def _():
        acc_ref[...] = jnp.zeros((G, D), jnp.float32)
    acc_ref[...] += part

    @pl.when(i == NBLK - 1)
    def _():
        cnt = jnp.maximum(cnt_ref[...], 1.0)
        pooled = jnp.maximum(acc_ref[...] / cnt, 0.0)
        hh = jnp.maximum(
            jnp.dot(pooled, wl1_ref[...],
                    preferred_element_type=jnp.float32) + bl1_ref[...], 0.0)
        o_ref[...] = jnp.dot(hh, wl2_ref[...],
                             preferred_element_type=jnp.float32) + bl2_ref[...]


def _tc3(z2, y2, dinv, b2, batch3, cnt, wl1, bl1, wl2, bl2):
    return pl.pallas_call(
        _tc3_body,
        grid=(NBLK,),
        in_specs=[
            pl.BlockSpec((BLK, D), lambda i: (i, 0)),
            pl.BlockSpec((BLK, D), lambda i: (i, 0)),
            pl.BlockSpec((BLK, 1), lambda i: (i, 0)),
            pl.BlockSpec((1, D), lambda i: (0, 0)),
            pl.BlockSpec((1, 1, BLK), lambda i: (i, 0, 0)),
            pl.BlockSpec((G, 1), lambda i: (0, 0)),
            pl.BlockSpec((D, 64), lambda i: (0, 0)),
            pl.BlockSpec((1, 64), lambda i: (0, 0)),
            pl.BlockSpec((64, 1), lambda i: (0, 0)),
            pl.BlockSpec((1, 1), lambda i: (0, 0)),
        ],
        out_specs=pl.BlockSpec((G, 1), lambda i: (0, 0)),
        out_shape=jax.ShapeDtypeStruct((G, 1), jnp.float32),
        scratch_shapes=[pltpu.VMEM((G, D), jnp.float32)],
    )(z2, y2, dinv, b2, batch3, cnt, wl1, bl1, wl2, bl2)


@jax.jit
def kernel(x, edge_index, edge_attr, batch, emb_table,
           W1, b1, W2, b2, Wl1, bl1, Wl2, bl2):
    del edge_attr
    src = edge_index[0]
    dst = edge_index[1]
    batch3 = batch.reshape(NBLK, 1, BLK)

    csrc, cdst_r, cnts, degp_raw = _route_call()(src, dst)
    degp = degp_raw[:, :N].reshape(NSC * NT, NBLK, BLK).transpose(1, 0, 2)

    y1, dinv, cnt = _tc1(x, emb_table, W1, degp, batch3)
    z1 = _conv_call()(y1.reshape(N * NSC, DH), csrc, cdst_r, cnts).reshape(N, D)
    y2 = _tc2(z1, y1, dinv, b1.reshape(1, D), W2)
    z2 = _conv_call()(y2.reshape(N * NSC, DH), csrc, cdst_r, cnts).reshape(N, D)
    out = _tc3(z2, y2, dinv, b2.reshape(1, D), batch3, cnt,
               Wl1, bl1.reshape(1, 64), Wl2, bl2.reshape(1, 1))
    return out[:, 0]
